# Initial kernel scaffold; baseline (speedup 1.0000x reference)
import jax
import jax.numpy as jnp
from jax.experimental import pallas as pl


def kernel(x, table):
    raise NotImplementedError("write your pallas kernel here")


# speedup vs baseline: 1.2661x; 1.2661x over previous
import functools

import jax
import jax.numpy as jnp
from jax import lax
from jax.experimental import pallas as pl
from jax.experimental.pallas import tpu as pltpu
from jax.experimental.pallas import tpu_sc as plsc

_NUM_CORES = 2
_NUM_SUBCORES = 16
_NW = _NUM_CORES * _NUM_SUBCORES

_CHUNK = 128
_D0 = 256
_DT = 128


def _build_gather(n_idx: int, vocab: int, dim: int):
    assert n_idx % (_NW * _CHUNK) == 0
    chunks_per_w = n_idx // (_NW * _CHUNK)

    mesh = plsc.VectorSubcoreMesh(
        core_axis_name="c", subcore_axis_name="s",
        num_cores=_NUM_CORES, num_subcores=_NUM_SUBCORES)

    @functools.partial(
        pl.kernel,
        out_type=(jax.ShapeDtypeStruct((n_idx, _D0), jnp.float32),
                  jax.ShapeDtypeStruct((n_idx, _DT), jnp.float32)),
        mesh=mesh,
        scratch_types=[
            pltpu.VMEM((chunks_per_w, _CHUNK), jnp.int32),
            pltpu.VMEM((_CHUNK, _D0), jnp.float32),
            pltpu.VMEM((_CHUNK, _DT), jnp.float32),
            pltpu.SemaphoreType.DMA,
            pltpu.SemaphoreType.DMA,
        ],
    )
    def gather(table_hbm, tail_hbm, idx_hbm, out_hbm, out2_hbm,
               idx_v, rows_v, tail_v, sem_a, sem_b):
        wid = lax.axis_index("s") * _NUM_CORES + lax.axis_index("c")
        cbase = wid * chunks_per_w
        pltpu.sync_copy(idx_hbm.at[wid], idx_v)

        @pl.loop(0, chunks_per_w)
        def _(c):
            cp_a = pltpu.async_copy(
                table_hbm.at[idx_v.at[c], pl.ds(0, _D0)], rows_v, sem_a)
            cp_b = pltpu.async_copy(tail_hbm.at[idx_v.at[c]], tail_v, sem_b)
            row0 = (cbase + c) * _CHUNK
            cp_a.wait()
            pltpu.sync_copy(rows_v, out_hbm.at[pl.ds(row0, _CHUNK)])
            cp_b.wait()
            pltpu.sync_copy(tail_v, out2_hbm.at[pl.ds(row0, _CHUNK)])

    return gather


def kernel(x, table):
    b, l = x.shape
    vocab, dim = table.shape
    n_idx = b * l
    idx = x.reshape(_NW, n_idx // (_NW * _CHUNK), _CHUNK).astype(jnp.int32)
    tail = jnp.pad(table[:, _D0:], ((0, 0), (0, _DT - (dim - _D0))))
    rows, rows_tail = _build_gather(n_idx, vocab, dim)(table, tail, idx)
    embeddings = jnp.concatenate(
        [rows, rows_tail[:, :dim - _D0]], axis=1).reshape(b, l, dim)
    mask = jnp.ones_like(x)
    return (embeddings, mask)
